# baseline (device time: 70443 ns/iter reference)
import jax
import jax.numpy as jnp
from jax import lax
from jax.experimental import pallas as pl
from jax.experimental.pallas import tpu as pltpu

N_DEV = 4


def kernel(ids, E):
    T = ids.shape[0]
    V_per, D = E.shape

    my = lax.axis_index("i")
    local = ids - my * V_per
    owned = (local >= 0) & (local < V_per)
    safe = jnp.where(owned, local, 0).astype(jnp.int32)
    own_i32 = owned.astype(jnp.int32)
    count = jnp.sum(own_i32, dtype=jnp.int32)[None]

    def body(safe_ref, own_ref, cnt_ref, e_ref, out_ref,
             pos_ref, lidx_ref, loc_sem, send_sem, recv_sem):
        my_pos = lax.axis_index("i")
        peers = [(my_pos + k) % N_DEV for k in range(1, N_DEV)]

        barrier_sem = pltpu.get_barrier_semaphore()
        for p in peers:
            pl.semaphore_signal(
                barrier_sem, inc=1,
                device_id=(p,), device_id_type=pl.DeviceIdType.MESH,
            )
        pl.semaphore_wait(barrier_sem, N_DEV - 1)

        def compact(t, k):
            pos_ref[k] = t
            lidx_ref[k] = safe_ref[t]
            return k + own_ref[t]

        lax.fori_loop(0, T, compact, 0)
        cnt = cnt_ref[0]

        def issue(i, carry):
            t = pos_ref[i]
            idx = lidx_ref[i]
            src = e_ref.at[pl.ds(idx, 1), :]
            dst = out_ref.at[pl.ds(t, 1), :]
            pltpu.make_async_copy(src, dst, loc_sem).start()
            for p in peers:
                pltpu.make_async_remote_copy(
                    src_ref=src,
                    dst_ref=dst,
                    send_sem=send_sem,
                    recv_sem=recv_sem,
                    device_id=(p,),
                    device_id_type=pl.DeviceIdType.MESH,
                ).start()
            return carry

        lax.fori_loop(0, cnt, issue, 0)

        dummy_src = e_ref.at[pl.ds(0, 1), :]
        dummy_dst = out_ref.at[pl.ds(0, 1), :]

        def remote_dummy():
            return pltpu.make_async_remote_copy(
                src_ref=dummy_src, dst_ref=dummy_dst,
                send_sem=send_sem, recv_sem=recv_sem,
                device_id=(peers[0],), device_id_type=pl.DeviceIdType.MESH,
            )

        def drain(n, wait_one):
            def do4(_, c):
                for _ in range(4):
                    wait_one()
                return c

            def do1(_, c):
                wait_one()
                return c

            lax.fori_loop(0, n // 4, do4, 0)
            lax.fori_loop(0, n % 4, do1, 0)

        drain(cnt, lambda: pltpu.make_async_copy(
            dummy_src, dummy_dst, loc_sem).wait())
        drain((N_DEV - 1) * cnt, lambda: remote_dummy().wait_send())
        drain(T - cnt, lambda: remote_dummy().wait_recv())

    return pl.pallas_call(
        body,
        out_shape=jax.ShapeDtypeStruct((T, D), jnp.float32),
        in_specs=[
            pl.BlockSpec(memory_space=pltpu.SMEM),
            pl.BlockSpec(memory_space=pltpu.SMEM),
            pl.BlockSpec(memory_space=pltpu.SMEM),
            pl.BlockSpec(memory_space=pltpu.MemorySpace.HBM),
        ],
        out_specs=pl.BlockSpec(memory_space=pltpu.VMEM),
        scratch_shapes=[
            pltpu.SMEM((2048,), jnp.int32),
            pltpu.SMEM((2048,), jnp.int32),
            pltpu.SemaphoreType.DMA,
            pltpu.SemaphoreType.DMA,
            pltpu.SemaphoreType.DMA,
        ],
        compiler_params=pltpu.CompilerParams(collective_id=0),
    )(safe, own_i32, count, E)
